# baseline (speedup 1.0000x reference)
import jax
import jax.numpy as jnp
from jax.experimental import pallas as pl

IN_DIM = 1024
OUT_DIM = 1024
N_PROTO = 64
RANK = 16
TOK_BLK = 128


def _x2_tree(x_ref):
    acc = None
    for c in range(8):
        xc = x_ref[:, c * 128:(c + 1) * 128]
        sqc = xc * xc
        acc = sqc if acc is None else acc + sqc
    t8 = acc[:, 0:8]
    for k in range(1, 16):
        t8 = t8 + acc[:, 8 * k:8 * k + 8]
    u = t8[:, 0:4] + t8[:, 4:8]
    v = u[:, 0:2] + u[:, 2:4]
    return v[:, 0:1] + v[:, 1:2]


def _body(x_ref, pt_ref, p2_ref, bt_ref, af_ref, bias_ref, temp_ref, exp_ref,
          o_ref):
    xb = x_ref[...]
    xb16 = xb.astype(jnp.bfloat16)
    logits = jnp.dot(xb16, pt_ref[...], preferred_element_type=jnp.float32)
    x2 = _x2_tree(x_ref)
    d2 = jnp.maximum((x2 + p2_ref[...]) - 2.0 * logits, 0.0)
    d = jnp.sqrt(d2)
    t = jnp.maximum(jnp.abs(temp_ref[0, 0]), 0.1)
    s = -d / t

    m = jnp.max(s, axis=1, keepdims=True)
    ex = jnp.exp(s - m)
    z = jnp.sum(ex, axis=1, keepdims=True)
    w = ex / z

    iota = jax.lax.broadcasted_iota(jnp.int32, w.shape, 1)
    m1 = jnp.max(w, axis=1, keepdims=True)
    i1 = jnp.min(jnp.where(w == m1, iota, N_PROTO), axis=1, keepdims=True)
    w_excl = jnp.where(iota == i1, -1.0, w)
    m2 = jnp.max(w_excl, axis=1, keepdims=True)
    i2 = jnp.min(jnp.where(w_excl == m2, iota, N_PROTO), axis=1, keepdims=True)
    sw = m1 + m2
    w1 = m1 / sw
    w2 = m2 / sw

    M = jnp.where(iota == i1, w1, 0.0) + jnp.where(iota == i2, w2, 0.0)

    H = jnp.dot(xb16, bt_ref[...], preferred_element_type=jnp.float32)
    Mexp = jnp.dot(M, exp_ref[...], preferred_element_type=jnp.float32)
    G = H * Mexp

    out = jnp.dot(G.astype(jnp.bfloat16), af_ref[...],
                  preferred_element_type=jnp.float32)
    out = out + jnp.dot(M, bias_ref[...], preferred_element_type=jnp.float32)
    o_ref[...] = out


def kernel(x, prototypes, B, A, bias, temp):
    lead_shape = x.shape[:-1]
    xf = x.reshape(-1, x.shape[-1])
    n_tok = xf.shape[0]

    pt = prototypes.T.astype(jnp.bfloat16)
    p2 = jnp.sum(prototypes * prototypes, axis=1)[None, :]
    bt = B.reshape(N_PROTO * RANK, IN_DIM).T.astype(jnp.bfloat16)
    af = A.transpose(0, 2, 1).reshape(N_PROTO * RANK, OUT_DIM).astype(jnp.bfloat16)
    temp_arr = jnp.asarray(temp, jnp.float32).reshape(1, 1)
    expand = (jnp.arange(N_PROTO)[:, None]
              == jnp.arange(N_PROTO * RANK)[None, :] // RANK).astype(jnp.float32)

    grid = (n_tok // TOK_BLK,)
    out = pl.pallas_call(
        _body,
        grid=grid,
        in_specs=[
            pl.BlockSpec((TOK_BLK, IN_DIM), lambda i: (i, 0)),
            pl.BlockSpec((IN_DIM, N_PROTO), lambda i: (0, 0)),
            pl.BlockSpec((1, N_PROTO), lambda i: (0, 0)),
            pl.BlockSpec((IN_DIM, N_PROTO * RANK), lambda i: (0, 0)),
            pl.BlockSpec((N_PROTO * RANK, OUT_DIM), lambda i: (0, 0)),
            pl.BlockSpec((N_PROTO, OUT_DIM), lambda i: (0, 0)),
            pl.BlockSpec((1, 1), lambda i: (0, 0)),
            pl.BlockSpec((N_PROTO, N_PROTO * RANK), lambda i: (0, 0)),
        ],
        out_specs=pl.BlockSpec((TOK_BLK, OUT_DIM), lambda i: (i, 0)),
        out_shape=jax.ShapeDtypeStruct((n_tok, OUT_DIM), jnp.float32),
    )(xf, pt, p2, bt, af, bias, temp_arr, expand)
    return out.reshape(*lead_shape, OUT_DIM)


# speedup vs baseline: 1.3782x; 1.3782x over previous
import jax
import jax.numpy as jnp
from jax.experimental import pallas as pl

IN_DIM = 1024
OUT_DIM = 1024
N_PROTO = 64
RANK = 16
TOK_BLK = 256


def _x2_tree(x_ref):
    acc = None
    for c in range(8):
        xc = x_ref[:, c * 128:(c + 1) * 128]
        sqc = xc * xc
        acc = sqc if acc is None else acc + sqc
    t8 = acc[:, 0:8]
    for k in range(1, 16):
        t8 = t8 + acc[:, 8 * k:8 * k + 8]
    u = t8[:, 0:4] + t8[:, 4:8]
    v = u[:, 0:2] + u[:, 2:4]
    return v[:, 0:1] + v[:, 1:2]


def _body(x_ref, pt_ref, p2_ref, bt_ref, af_ref, bias_ref, temp_ref, exp_ref,
          o_ref):
    xb = x_ref[...]
    xb16 = xb.astype(jnp.bfloat16)
    logits = jnp.dot(xb16, pt_ref[...], preferred_element_type=jnp.float32)
    x2 = _x2_tree(x_ref)
    d2 = jnp.maximum((x2 + p2_ref[...]) - 2.0 * logits, 0.0)
    d = jnp.sqrt(d2)
    t = jnp.maximum(jnp.abs(temp_ref[0, 0]), 0.1)
    s = -d / t

    m = jnp.max(s, axis=1, keepdims=True)
    ex = jnp.exp(s - m)
    z = jnp.sum(ex, axis=1, keepdims=True)
    w = ex / z

    iota = jax.lax.broadcasted_iota(jnp.int32, w.shape, 1)
    m1 = jnp.max(w, axis=1, keepdims=True)
    i1 = jnp.min(jnp.where(w == m1, iota, N_PROTO), axis=1, keepdims=True)
    w_excl = jnp.where(iota == i1, -1.0, w)
    m2 = jnp.max(w_excl, axis=1, keepdims=True)
    i2 = jnp.min(jnp.where(w_excl == m2, iota, N_PROTO), axis=1, keepdims=True)
    sw = m1 + m2
    w1 = m1 / sw
    w2 = m2 / sw

    M = jnp.where(iota == i1, w1, 0.0) + jnp.where(iota == i2, w2, 0.0)

    H = jnp.dot(xb16, bt_ref[...], preferred_element_type=jnp.float32)
    Mexp = jnp.dot(M, exp_ref[...], preferred_element_type=jnp.float32)
    G = H * Mexp

    out = jnp.dot(G.astype(jnp.bfloat16), af_ref[...],
                  preferred_element_type=jnp.float32)
    out = out + jnp.dot(M, bias_ref[...], preferred_element_type=jnp.float32)
    o_ref[...] = out


def kernel(x, prototypes, B, A, bias, temp):
    lead_shape = x.shape[:-1]
    xf = x.reshape(-1, x.shape[-1])
    n_tok = xf.shape[0]

    pt = prototypes.T.astype(jnp.bfloat16)
    p2 = jnp.sum(prototypes * prototypes, axis=1)[None, :]
    bt = B.reshape(N_PROTO * RANK, IN_DIM).T.astype(jnp.bfloat16)
    af = A.transpose(0, 2, 1).reshape(N_PROTO * RANK, OUT_DIM).astype(jnp.bfloat16)
    temp_arr = jnp.asarray(temp, jnp.float32).reshape(1, 1)
    expand = (jnp.arange(N_PROTO)[:, None]
              == jnp.arange(N_PROTO * RANK)[None, :] // RANK).astype(jnp.float32)

    grid = (n_tok // TOK_BLK,)
    out = pl.pallas_call(
        _body,
        grid=grid,
        in_specs=[
            pl.BlockSpec((TOK_BLK, IN_DIM), lambda i: (i, 0)),
            pl.BlockSpec((IN_DIM, N_PROTO), lambda i: (0, 0)),
            pl.BlockSpec((1, N_PROTO), lambda i: (0, 0)),
            pl.BlockSpec((IN_DIM, N_PROTO * RANK), lambda i: (0, 0)),
            pl.BlockSpec((N_PROTO * RANK, OUT_DIM), lambda i: (0, 0)),
            pl.BlockSpec((N_PROTO, OUT_DIM), lambda i: (0, 0)),
            pl.BlockSpec((1, 1), lambda i: (0, 0)),
            pl.BlockSpec((N_PROTO, N_PROTO * RANK), lambda i: (0, 0)),
        ],
        out_specs=pl.BlockSpec((TOK_BLK, OUT_DIM), lambda i: (i, 0)),
        out_shape=jax.ShapeDtypeStruct((n_tok, OUT_DIM), jnp.float32),
    )(xf, pt, p2, bt, af, bias, temp_arr, expand)
    return out.reshape(*lead_shape, OUT_DIM)
